# Initial kernel scaffold; baseline (speedup 1.0000x reference)
import jax
import jax.numpy as jnp
from jax.experimental import pallas as pl


def kernel(x):
    raise NotImplementedError("write your pallas kernel here")


# speedup vs baseline: 1.1757x; 1.1757x over previous
import jax
import jax.numpy as jnp
from jax.experimental import pallas as pl
from jax.experimental.pallas import tpu as pltpu

_B, _R, _C = 4, 4096, 2048
_CBLK = 512


def _argmin_body(x_ref, o_ref):
    v = x_ref[0]
    m = jnp.min(v, axis=0, keepdims=True)
    iota = jax.lax.broadcasted_iota(jnp.int32, v.shape, 0)
    idx = jnp.min(jnp.where(v <= m, iota, _R), axis=0)
    o_ref[0, 0] = idx


def kernel(x):
    out = pl.pallas_call(
        _argmin_body,
        grid=(_B, _C // _CBLK),
        in_specs=[pl.BlockSpec((1, _R, _CBLK), lambda b, c: (b, 0, c))],
        out_specs=pl.BlockSpec((1, 1, _CBLK), lambda b, c: (b, 0, c)),
        out_shape=jax.ShapeDtypeStruct((_B, 1, _C), jnp.int32),
        compiler_params=pltpu.CompilerParams(
            dimension_semantics=("parallel", "parallel"),
        ),
    )(x)
    return out.reshape(_B, _C).astype(jnp.int64)
